# Initial kernel scaffold; baseline (speedup 1.0000x reference)
import jax
import jax.numpy as jnp
from jax.experimental import pallas as pl


def kernel(input, param):
    raise NotImplementedError("write your pallas kernel here")


# speedup vs baseline: 1.3197x; 1.3197x over previous
import functools

import jax
import jax.numpy as jnp
from jax import lax
from jax.experimental import pallas as pl
from jax.experimental.pallas import tpu as pltpu
from jax.experimental.pallas import tpu_sc as plsc

N_ROWS = 1_000_000
DIM = 64
BATCH = 819_200

NUM_CORES = 2
NUM_SUBCORES = 16
LANES = 16
NUM_WORKERS = NUM_CORES * NUM_SUBCORES

B_PER_W = BATCH // NUM_WORKERS
CHUNK = 512
SUB = 128
KSUB = CHUNK // SUB
NUM_CHUNKS = B_PER_W // CHUNK


def _sampler_body(table_hbm, param_hbm, out_hbm,
                  param_v, w_v, idx0_v, idx1_v, rows0_v, rows1_v, sem):
    wid = lax.axis_index("s") * NUM_CORES + lax.axis_index("c")
    base = wid * B_PER_W
    scale = jnp.float32(N_ROWS - 1)

    def chunk_body(g, carry):
        off = base + g * CHUNK
        pltpu.sync_copy(param_hbm.at[pl.ds(off, CHUNK)], param_v)

        for j in range(CHUNK // LANES):
            p = param_v[pl.ds(j * LANES, LANES)]
            t = jnp.minimum(jnp.maximum(p, 0.0), 1.0) * scale
            i0 = t.astype(jnp.int32)
            i1 = jnp.minimum(i0 + 1, N_ROWS - 1)
            w = t - i0.astype(jnp.float32)
            k, r = divmod(j * LANES, SUB)
            idx0_v[k, pl.ds(r, LANES)] = i0
            idx1_v[k, pl.ds(r, LANES)] = i1
            w_v[pl.ds(j * LANES, LANES)] = w

        copies = []
        for k in range(KSUB):
            copies.append(
                pltpu.async_copy(table_hbm.at[idx0_v.at[k]], rows0_v.at[k], sem))
            copies.append(
                pltpu.async_copy(table_hbm.at[idx1_v.at[k]], rows1_v.at[k], sem))
        for cp in copies:
            cp.wait()

        for k in range(KSUB):
            def row16_body(r16, c, _k=k):
                w16 = w_v[pl.ds(_k * SUB + r16 * LANES, LANES)]
                for j in range(LANES):
                    wb = w16.at[jnp.full((LANES,), j, jnp.int32)].get(
                        mode="promise_in_bounds")
                    one_m = 1.0 - wb
                    r = r16 * LANES + j
                    for cc in range(DIM // LANES):
                        v0 = rows0_v[_k, r, pl.ds(cc * LANES, LANES)]
                        v1 = rows1_v[_k, r, pl.ds(cc * LANES, LANES)]
                        rows0_v[_k, r, pl.ds(cc * LANES, LANES)] = (
                            v0 * one_m + v1 * wb)
                return c
            lax.fori_loop(0, SUB // LANES, row16_body, 0)

        for k in range(KSUB):
            pltpu.sync_copy(rows0_v.at[k],
                            out_hbm.at[pl.ds(off + k * SUB, SUB)])
        return carry

    lax.fori_loop(0, NUM_CHUNKS, chunk_body, 0)


@jax.jit
def kernel(input, param):
    mesh = plsc.VectorSubcoreMesh(core_axis_name="c", subcore_axis_name="s")
    f = pl.kernel(
        _sampler_body,
        out_type=jax.ShapeDtypeStruct((BATCH, DIM), jnp.float32),
        mesh=mesh,
        scratch_types=[
            pltpu.VMEM((CHUNK,), jnp.float32),
            pltpu.VMEM((CHUNK,), jnp.float32),
            pltpu.VMEM((KSUB, SUB), jnp.int32),
            pltpu.VMEM((KSUB, SUB), jnp.int32),
            pltpu.VMEM((KSUB, SUB, DIM), jnp.float32),
            pltpu.VMEM((KSUB, SUB, DIM), jnp.float32),
            pltpu.SemaphoreType.DMA,
        ],
        compiler_params=pltpu.CompilerParams(use_tc_tiling_on_sc=False),
    )
    return f(input, param)
